# baseline (speedup 1.0000x reference)
import functools

import jax
import jax.numpy as jnp
from jax import lax
from jax.experimental import pallas as pl
from jax.experimental.pallas import tpu as pltpu
from jax.experimental.pallas import tpu_sc as plsc


def _make_sc_gather(n_rows, emb):
  info = plsc.get_sparse_core_info()
  nc, ns = info.num_cores, info.num_subcores
  nw = nc * ns
  b_per_w = n_rows // nw
  ch = 128
  n_ch = b_per_w // ch
  fire = max(f for f in range(1, min(16, n_ch) + 1) if n_ch % f == 0)
  mesh = plsc.VectorSubcoreMesh(core_axis_name="c", subcore_axis_name="s")

  @functools.partial(
      pl.kernel, mesh=mesh,
      out_type=jax.ShapeDtypeStruct((n_rows, emb), jnp.float32),
      compiler_params=pltpu.CompilerParams(use_tc_tiling_on_sc=False),
      scratch_types=[
          pltpu.VMEM((n_ch, ch), jnp.int32),
          pltpu.VMEM((fire * ch, emb), jnp.float32),
          pltpu.SemaphoreType.DMA,
      ],
  )
  def gather(table_hbm, idx_hbm, out_hbm, idx_v, rows_v, sem):
    wid = lax.axis_index("s") * nc + lax.axis_index("c")
    base = wid * b_per_w
    pltpu.sync_copy(idx_hbm.at[wid], idx_v)

    def round_(r):
      for k in range(fire):
        pltpu.async_copy(table_hbm.at[idx_v.at[r * fire + k]],
                         rows_v.at[pl.ds(k * ch, ch)], sem)
      for k in range(fire):
        pltpu.make_async_copy(table_hbm.at[idx_v.at[0]],
                              rows_v.at[pl.ds(k * ch, ch)], sem).wait()
      pltpu.sync_copy(rows_v,
                      out_hbm.at[pl.ds(base + r * fire * ch, fire * ch)])

    pl.loop(0, n_ch // fire)(round_)

  return gather, nw, n_ch, ch


def _lm_head(xt3, posk, wt, bcol, b_blk, t_total, t_off, prev=None):
  t_blk, n_b, emb = xt3.shape
  vocab = wt.shape[0]
  grid = (n_b // b_blk,)

  def body(*refs):
    x_ref, p_ref, w_ref, b_ref = refs[-5:-1]
    o_ref = refs[-1]
    for t in range(t_blk):
      xt = x_ref[t] + p_ref[t, :][None, :]
      o_ref[t] = (
          jax.lax.dot_general(
              w_ref[...], xt, (((1,), (1,)), ((), ())),
              preferred_element_type=jnp.float32)
          + b_ref[...]
      )

  in_specs = [
      pl.BlockSpec((t_blk, b_blk, emb), lambda i: (0, i, 0)),
      pl.BlockSpec((t_blk, emb), lambda i: (0, 0)),
      pl.BlockSpec((vocab, emb), lambda i: (0, 0)),
      pl.BlockSpec((vocab, 1), lambda i: (0, 0)),
  ]
  args = [xt3, posk, wt, bcol]
  aliases = {}
  if prev is not None:
    in_specs = [pl.BlockSpec(memory_space=pl.ANY)] + in_specs
    args = [prev] + args
    aliases = {0: 0}

  return pl.pallas_call(
      body,
      grid=grid,
      in_specs=in_specs,
      out_specs=pl.BlockSpec((t_blk, vocab, b_blk),
                             lambda i: (t_off, 0, i)),
      out_shape=jax.ShapeDtypeStruct((t_total, vocab, n_b), jnp.float32),
      input_output_aliases=aliases,
      compiler_params=pltpu.CompilerParams(
          dimension_semantics=("arbitrary",),
      ),
  )(*args)


def kernel(idx, tok_table, pos_table, W, b):
  B, T = idx.shape
  vocab, emb = tok_table.shape
  b_blk = 512
  t_blk = T // 2

  idxT = idx.T
  wt, bcol = W.T, b.reshape(vocab, 1)
  out = None
  for k in range(2):
    sc_gather, nw, n_ch, ch = _make_sc_gather(t_blk * B, emb)
    idx_c = idxT[k * t_blk:(k + 1) * t_blk].reshape(nw, n_ch, ch)
    x_c = sc_gather(tok_table, idx_c).reshape(t_blk, B, emb)
    posk = pos_table[k * t_blk:(k + 1) * t_blk]
    out = _lm_head(x_c, posk, wt, bcol, b_blk, T, k, prev=out)
  return out.transpose(2, 0, 1)


# speedup vs baseline: 1.2648x; 1.2648x over previous
import functools

import jax
import jax.numpy as jnp
from jax import lax
from jax.experimental import pallas as pl
from jax.experimental.pallas import tpu as pltpu
from jax.experimental.pallas import tpu_sc as plsc


def _make_sc_gather(t_blk, n_b, emb):
  info = plsc.get_sparse_core_info()
  nc, ns = info.num_cores, info.num_subcores
  nw = nc * ns
  n_rows = t_blk * n_b
  b_per_w = n_rows // nw
  ch = 128
  n_ch = b_per_w // ch
  fire = max(f for f in range(1, min(16, n_ch) + 1) if n_ch % f == 0)
  w_per_t = nw // t_blk
  n_j = n_b // 4
  mesh = plsc.VectorSubcoreMesh(core_axis_name="c", subcore_axis_name="s")

  @functools.partial(
      pl.kernel, mesh=mesh,
      out_type=jax.ShapeDtypeStruct((t_blk, n_j, 4 * emb), jnp.float32),
      compiler_params=pltpu.CompilerParams(use_tc_tiling_on_sc=False),
      scratch_types=[
          pltpu.VMEM((n_ch, ch), jnp.int32),
          pltpu.VMEM((fire * ch, emb), jnp.float32),
          pltpu.SemaphoreType.DMA,
      ],
  )
  def gather(table_hbm, idx_hbm, out_hbm, idx_v, rows_v, sem):
    wid = lax.axis_index("s") * nc + lax.axis_index("c")
    t = wid // w_per_t
    m = wid % w_per_t
    b0 = m * b_per_w
    r = b0 // n_j
    j00 = b0 % n_j
    pltpu.sync_copy(idx_hbm.at[wid], idx_v)

    def round_(rd):
      for k in range(fire):
        pltpu.async_copy(table_hbm.at[idx_v.at[rd * fire + k]],
                         rows_v.at[pl.ds(k * ch, ch)], sem)
      for k in range(fire):
        pltpu.make_async_copy(table_hbm.at[idx_v.at[0]],
                              rows_v.at[pl.ds(k * ch, ch)], sem).wait()
      pltpu.sync_copy(
          rows_v,
          out_hbm.at[t, pl.ds(j00 + rd * fire * ch, fire * ch),
                     pl.ds(r * emb, emb)])

    pl.loop(0, n_ch // fire)(round_)

  return gather, nw, n_ch, ch


def _lm_head(xp, poswb, wt, b_blk, t_total, t_off, prev=None):
  t_blk, n_j, _ = xp.shape
  vocab, emb = wt.shape
  grid = (n_j // b_blk, 4)

  def body(*refs):
    x_ref, w_ref, pw_ref = refs[-4:-1]
    o_ref = refs[-1]
    r = pl.program_id(1)
    for rr in range(4):
      @pl.when(r == rr)
      def _():
        for t in range(t_blk):
          p = x_ref[t][:, rr * emb:(rr + 1) * emb]
          o_ref[t] = (
              jax.lax.dot_general(
                  w_ref[...], p, (((1,), (1,)), ((), ())),
                  preferred_element_type=jnp.float32)
              + pw_ref[t][:, None]
          )

  in_specs = [
      pl.BlockSpec((t_blk, b_blk, 4 * emb), lambda j, r: (0, j, 0)),
      pl.BlockSpec((vocab, emb), lambda j, r: (0, 0)),
      pl.BlockSpec((t_blk, vocab), lambda j, r: (0, 0)),
  ]
  args = [xp, wt, poswb]
  aliases = {}
  if prev is not None:
    in_specs = [pl.BlockSpec(memory_space=pl.ANY)] + in_specs
    args = [prev] + args
    aliases = {0: 0}

  n_jb = n_j // b_blk
  return pl.pallas_call(
      body,
      grid=grid,
      in_specs=in_specs,
      out_specs=pl.BlockSpec((t_blk, vocab, b_blk),
                             lambda j, r: (t_off, 0, r * n_jb + j)),
      out_shape=jax.ShapeDtypeStruct((t_total, vocab, 4 * n_j), jnp.float32),
      input_output_aliases=aliases,
      compiler_params=pltpu.CompilerParams(
          dimension_semantics=("arbitrary", "arbitrary"),
      ),
  )(*args)


def kernel(idx, tok_table, pos_table, W, b):
  B, T = idx.shape
  vocab, emb = tok_table.shape
  b_blk = 512
  t_blk = T // 2

  idxT = idx.T
  wt = W.T
  poswb = pos_table @ W + b[None, :]
  out = None
  for k in range(2):
    sc_gather, nw, n_ch, ch = _make_sc_gather(t_blk, B, emb)
    idx_c = idxT[k * t_blk:(k + 1) * t_blk].reshape(nw, n_ch, ch)
    xp = sc_gather(tok_table, idx_c)
    out = _lm_head(xp, poswb[k * t_blk:(k + 1) * t_blk], wt,
                   b_blk, T, k, prev=out)
  return out.transpose(2, 0, 1)
